# baseline (device time: 250303 ns/iter reference)
import jax
import jax.numpy as jnp
from jax import lax
from jax.experimental import pallas as pl
from jax.experimental.pallas import tpu as pltpu

BLK = 512

_CompilerParams = getattr(pltpu, "CompilerParams", None) or getattr(
    pltpu, "TPUCompilerParams"
)
_DeviceIdType = getattr(pl, "DeviceIdType", None) or getattr(pltpu, "DeviceIdType")
_sem_signal = getattr(pl, "semaphore_signal", None) or getattr(
    pltpu, "semaphore_signal"
)
_sem_wait = getattr(pl, "semaphore_wait", None) or getattr(pltpu, "semaphore_wait")
_HBM = pltpu.MemorySpace.HBM


def kernel(O, Wo):
    B, S, H, D = O.shape
    K = H * D
    N = Wo.shape[1]
    R = S // 2
    NB = N // BLK

    O2 = O.reshape(S, K).astype(jnp.bfloat16)

    def body(o_ref, wo_ref, out_mine_ref, out_recv_ref, send_buf,
             send_sems, recv_sems):
        t = pl.program_id(0)
        x = lax.axis_index("x")
        y = lax.axis_index("y")
        z = lax.axis_index("z")
        peer = (1 - x, y, z)
        is0 = x == 0

        @pl.when(t == 0)
        def _():
            barrier = pltpu.get_barrier_semaphore()
            _sem_signal(barrier, inc=1, device_id=peer,
                        device_id_type=_DeviceIdType.MESH)
            _sem_wait(barrier, 1)

        wt = wo_ref[...].astype(jnp.bfloat16)
        p = jnp.dot(o_ref[...], wt, preferred_element_type=jnp.float32)
        mine = jnp.where(is0, p[:R, :], p[R:, :])
        their = jnp.where(is0, p[R:, :], p[:R, :])

        out_mine_ref[...] = mine.astype(jnp.bfloat16)
        send_buf[t] = their.astype(jnp.bfloat16)

        def rdma(i):
            return pltpu.make_async_remote_copy(
                src_ref=send_buf.at[i],
                dst_ref=out_recv_ref.at[:, pl.ds(i * BLK, BLK)],
                send_sem=send_sems.at[i],
                recv_sem=recv_sems.at[i],
                device_id=peer,
                device_id_type=_DeviceIdType.MESH,
            )

        rdma(t).start()

        @pl.when(t == NB - 1)
        def _():
            for i in range(NB):
                rdma(i).wait_send()
            for i in range(NB):
                rdma(i).wait_recv()

    out_mine, out_recv = pl.pallas_call(
        body,
        grid=(NB,),
        out_shape=[
            jax.ShapeDtypeStruct((R, N), jnp.bfloat16),
            jax.ShapeDtypeStruct((R, N), jnp.bfloat16),
        ],
        in_specs=[
            pl.BlockSpec((S, K), lambda t: (0, 0)),
            pl.BlockSpec((K, BLK), lambda t: (0, t)),
        ],
        out_specs=[
            pl.BlockSpec((R, BLK), lambda t: (0, t)),
            pl.BlockSpec(memory_space=_HBM),
        ],
        scratch_shapes=[
            pltpu.VMEM((NB, R, BLK), jnp.bfloat16),
            pltpu.SemaphoreType.DMA((NB,)),
            pltpu.SemaphoreType.DMA((NB,)),
        ],
        compiler_params=_CompilerParams(
            dimension_semantics=("arbitrary",),
            collective_id=0,
            vmem_limit_bytes=60 * 1024 * 1024,
        ),
    )(O2, Wo)

    out = out_mine.astype(jnp.float32) + out_recv.astype(jnp.float32)
    return out.reshape(B, R, N)


# device time: 244585 ns/iter; 1.0234x vs baseline; 1.0234x over previous
import jax
import jax.numpy as jnp
from jax import lax
from jax.experimental import pallas as pl
from jax.experimental.pallas import tpu as pltpu

BLK = 256

_CompilerParams = getattr(pltpu, "CompilerParams", None) or getattr(
    pltpu, "TPUCompilerParams"
)
_DeviceIdType = getattr(pl, "DeviceIdType", None) or getattr(pltpu, "DeviceIdType")
_sem_signal = getattr(pl, "semaphore_signal", None) or getattr(
    pltpu, "semaphore_signal"
)
_sem_wait = getattr(pl, "semaphore_wait", None) or getattr(pltpu, "semaphore_wait")
_HBM = pltpu.MemorySpace.HBM


def kernel(O, Wo):
    B, S, H, D = O.shape
    K = H * D
    N = Wo.shape[1]
    R = S // 2
    NB = N // BLK

    O2 = O.reshape(S, K).astype(jnp.bfloat16)

    def body(o_ref, wo_ref, out_mine_ref, out_recv_ref, send_buf,
             send_sems, recv_sems):
        t = pl.program_id(0)
        x = lax.axis_index("x")
        y = lax.axis_index("y")
        z = lax.axis_index("z")
        peer = (1 - x, y, z)
        is0 = x == 0

        @pl.when(t == 0)
        def _():
            barrier = pltpu.get_barrier_semaphore()
            _sem_signal(barrier, inc=1, device_id=peer,
                        device_id_type=_DeviceIdType.MESH)
            _sem_wait(barrier, 1)

        wt = wo_ref[...].astype(jnp.bfloat16)
        p = jnp.dot(o_ref[...], wt, preferred_element_type=jnp.float32)
        p_top = p[:R, :].astype(jnp.bfloat16)
        p_bot = p[R:, :].astype(jnp.bfloat16)

        @pl.when(is0)
        def _():
            out_mine_ref[...] = p_top
            send_buf[t] = p_bot

        @pl.when(jnp.logical_not(is0))
        def _():
            out_mine_ref[...] = p_bot
            send_buf[t] = p_top

        def rdma(i):
            return pltpu.make_async_remote_copy(
                src_ref=send_buf.at[i],
                dst_ref=out_recv_ref.at[:, pl.ds(i * BLK, BLK)],
                send_sem=send_sems.at[i],
                recv_sem=recv_sems.at[i],
                device_id=peer,
                device_id_type=_DeviceIdType.MESH,
            )

        rdma(t).start()

        @pl.when(t == NB - 1)
        def _():
            for i in range(NB):
                rdma(i).wait_send()
            for i in range(NB):
                rdma(i).wait_recv()

    out_mine, out_recv = pl.pallas_call(
        body,
        grid=(NB,),
        out_shape=[
            jax.ShapeDtypeStruct((R, N), jnp.bfloat16),
            jax.ShapeDtypeStruct((R, N), jnp.bfloat16),
        ],
        in_specs=[
            pl.BlockSpec(memory_space=pltpu.MemorySpace.VMEM),
            pl.BlockSpec((K, BLK), lambda t: (0, t)),
        ],
        out_specs=[
            pl.BlockSpec((R, BLK), lambda t: (0, t)),
            pl.BlockSpec(memory_space=_HBM),
        ],
        scratch_shapes=[
            pltpu.VMEM((NB, R, BLK), jnp.bfloat16),
            pltpu.SemaphoreType.DMA((NB,)),
            pltpu.SemaphoreType.DMA((NB,)),
        ],
        compiler_params=_CompilerParams(
            dimension_semantics=("arbitrary",),
            collective_id=0,
            vmem_limit_bytes=60 * 1024 * 1024,
        ),
    )(O2, Wo)

    out = out_mine.astype(jnp.float32) + out_recv.astype(jnp.float32)
    return out.reshape(B, R, N)


# device time: 234326 ns/iter; 1.0682x vs baseline; 1.0438x over previous
import jax
import jax.numpy as jnp
from jax import lax
from jax.experimental import pallas as pl
from jax.experimental.pallas import tpu as pltpu

BLK = 256
LAG = 8
SEND_SLOTS = 8

_CompilerParams = getattr(pltpu, "CompilerParams", None) or getattr(
    pltpu, "TPUCompilerParams"
)
_DeviceIdType = getattr(pl, "DeviceIdType", None) or getattr(pltpu, "DeviceIdType")
_sem_signal = getattr(pl, "semaphore_signal", None) or getattr(
    pltpu, "semaphore_signal"
)
_sem_wait = getattr(pl, "semaphore_wait", None) or getattr(pltpu, "semaphore_wait")


def kernel(O, Wo):
    B, S, H, D = O.shape
    K = H * D
    N = Wo.shape[1]
    R = S // 2
    NB = N // BLK
    RING = LAG + 1

    O2 = O.reshape(S, K).astype(jnp.bfloat16)

    def body(o_ref, wo_ref, out_ref, send_buf, recv_buf, mine_ring,
             send_sems, recv_sems):
        t = pl.program_id(0)
        x = lax.axis_index("x")
        y = lax.axis_index("y")
        z = lax.axis_index("z")
        peer = (1 - x, y, z)
        is0 = x == 0

        def rdma(slot, chunk):
            return pltpu.make_async_remote_copy(
                src_ref=send_buf.at[slot],
                dst_ref=recv_buf.at[chunk],
                send_sem=send_sems.at[slot],
                recv_sem=recv_sems.at[chunk],
                device_id=peer,
                device_id_type=_DeviceIdType.MESH,
            )

        @pl.when(t == 0)
        def _():
            barrier = pltpu.get_barrier_semaphore()
            _sem_signal(barrier, inc=1, device_id=peer,
                        device_id_type=_DeviceIdType.MESH)
            _sem_wait(barrier, 1)

        @pl.when(t < NB)
        def _():
            wt = wo_ref[...].astype(jnp.bfloat16)
            p = jnp.dot(o_ref[...], wt, preferred_element_type=jnp.float32)
            p_top = p[:R, :].astype(jnp.bfloat16)
            p_bot = p[R:, :].astype(jnp.bfloat16)

            slot = t % SEND_SLOTS

            @pl.when(t >= SEND_SLOTS)
            def _():
                rdma(slot, 0).wait_send()

            ring = t % RING

            @pl.when(is0)
            def _():
                mine_ring[ring] = p_top
                send_buf[slot] = p_bot

            @pl.when(jnp.logical_not(is0))
            def _():
                mine_ring[ring] = p_bot
                send_buf[slot] = p_top

            rdma(slot, t).start()

        @pl.when(t >= LAG)
        def _():
            c = t - LAG
            rdma(0, c).wait_recv()
            out_ref[0] = (
                mine_ring[c % RING].astype(jnp.float32)
                + recv_buf[c].astype(jnp.float32)
            ).astype(jnp.bfloat16)

        @pl.when(t == NB + LAG - 1)
        def _():
            for i in range(SEND_SLOTS):
                rdma(i, 0).wait_send()

    out = pl.pallas_call(
        body,
        grid=(NB + LAG,),
        out_shape=jax.ShapeDtypeStruct((B, R, N), jnp.bfloat16),
        in_specs=[
            pl.BlockSpec(memory_space=pltpu.MemorySpace.VMEM),
            pl.BlockSpec((K, BLK), lambda t: (0, jnp.minimum(t, NB - 1))),
        ],
        out_specs=pl.BlockSpec(
            (B, R, BLK), lambda t: (0, 0, jnp.clip(t - LAG, 0, NB - 1))
        ),
        scratch_shapes=[
            pltpu.VMEM((SEND_SLOTS, R, BLK), jnp.bfloat16),
            pltpu.VMEM((NB, R, BLK), jnp.bfloat16),
            pltpu.VMEM((RING, R, BLK), jnp.bfloat16),
            pltpu.SemaphoreType.DMA((SEND_SLOTS,)),
            pltpu.SemaphoreType.DMA((NB,)),
        ],
        compiler_params=_CompilerParams(
            dimension_semantics=("arbitrary",),
            collective_id=0,
            vmem_limit_bytes=int(62.9 * 1024 * 1024),
        ),
    )(O2, Wo)

    return out
